# baseline (speedup 1.0000x reference)
import functools

import jax
import jax.numpy as jnp
from jax import lax
from jax.experimental import pallas as pl
from jax.experimental.pallas import tpu as pltpu
from jax.experimental.pallas import tpu_sc as plsc

B = 1024
LQ = 20
LC = 200
WE = 64
QE = 64
KS = 3
TOPN = 10

NC = 2
NS = 16
NW = NC * NS

FIRE = 16
SCHUNK = 320
HB = B // 2
C_PER_W = (HB * LC) // NW
Q_PER_W = (B * LQ) // NW
C_CHUNKS = C_PER_W // SCHUNK
QCHUNK = 128
QN_CHUNKS = Q_PER_W // QCHUNK
NBATCH = SCHUNK // FIRE
QNBATCH = QCHUNK // FIRE


def _sc_gather_kernel_cq(cand_idx_hbm, q_idx_hbm, table_hbm,
                         cand_out_hbm, q_out_hbm,
                         idx_c_v, idx_q_v, rows_v, qrows_v, gsem, wsem):
    _sc_gather_body(cand_idx_hbm, q_idx_hbm, table_hbm, cand_out_hbm,
                    q_out_hbm, idx_c_v, idx_q_v, rows_v, qrows_v, gsem, wsem)


def _sc_gather_kernel_c(cand_idx_hbm, table_hbm, cand_out_hbm,
                        idx_c_v, rows_v, gsem, wsem):
    _sc_gather_body(cand_idx_hbm, None, table_hbm, cand_out_hbm,
                    None, idx_c_v, None, rows_v, None, gsem, wsem)


def _sc_gather_body(cand_idx_hbm, q_idx_hbm, table_hbm,
                    cand_out_hbm, q_out_hbm,
                    idx_c_v, idx_q_v, rows_v, qrows_v, gsem, wsem):
    wid = lax.axis_index("s") * NC + lax.axis_index("c")
    base_c = wid * C_PER_W
    base_q = wid * Q_PER_W

    pltpu.sync_copy(cand_idx_hbm.at[wid], idx_c_v)
    if q_idx_hbm is not None:
        pltpu.sync_copy(q_idx_hbm.at[wid], idx_q_v)

    def fire_chunk(idx_ref, buf, nbatch, sem):
        def body(g, c):
            vec = idx_ref[pl.ds(g * FIRE, FIRE)]
            for u in range(FIRE):
                pltpu.async_copy(table_hbm.at[pl.ds(vec[u], 1)],
                                 buf.at[pl.ds(g * FIRE + u, 1)], sem)
            return c
        lax.fori_loop(0, nbatch, body, 0)

    def drain_chunk(buf, nrows, sem):
        def body(g, c):
            for u in range(FIRE):
                pltpu.make_async_copy(table_hbm.at[pl.ds(0, 1)],
                                      buf.at[pl.ds(g * FIRE + u, 1)], sem).wait()
            return c
        lax.fori_loop(0, nrows // FIRE, body, 0)

    def chunk_body(ch, carry):
        @pl.when(ch >= 2)
        def _():
            pltpu.make_async_copy(
                rows_v.at[0],
                cand_out_hbm.at[pl.ds(base_c, SCHUNK)], wsem).wait()
        fire_chunk(idx_c_v.at[ch], rows_v.at[ch % 2], NBATCH, gsem.at[ch % 2])
        prev = ch - 1
        drain_chunk(rows_v.at[prev % 2], SCHUNK, gsem.at[prev % 2])
        pltpu.async_copy(rows_v.at[prev % 2],
                         cand_out_hbm.at[pl.ds(base_c + prev * SCHUNK, SCHUNK)],
                         wsem)
        return carry

    fire_chunk(idx_c_v.at[0], rows_v.at[0], NBATCH, gsem.at[0])
    lax.fori_loop(1, C_CHUNKS, chunk_body, 0)
    last = C_CHUNKS - 1
    drain_chunk(rows_v.at[last % 2], SCHUNK, gsem.at[last % 2])
    pltpu.async_copy(rows_v.at[last % 2],
                     cand_out_hbm.at[pl.ds(base_c + last * SCHUNK, SCHUNK)], wsem)
    pltpu.make_async_copy(rows_v.at[0], cand_out_hbm.at[pl.ds(base_c, SCHUNK)],
                          wsem).wait()
    pltpu.make_async_copy(rows_v.at[0], cand_out_hbm.at[pl.ds(base_c, SCHUNK)],
                          wsem).wait()

    if q_idx_hbm is None:
        return

    def q_chunk_body(ch, carry):
        @pl.when(ch >= 2)
        def _():
            pltpu.make_async_copy(
                qrows_v.at[0],
                q_out_hbm.at[pl.ds(base_q, QCHUNK)], wsem).wait()
        fire_chunk(idx_q_v.at[ch], qrows_v.at[ch % 2], QNBATCH, gsem.at[ch % 2])
        prev = ch - 1
        drain_chunk(qrows_v.at[prev % 2], QCHUNK, gsem.at[prev % 2])
        pltpu.async_copy(qrows_v.at[prev % 2],
                         q_out_hbm.at[pl.ds(base_q + prev * QCHUNK, QCHUNK)], wsem)
        return carry

    fire_chunk(idx_q_v.at[0], qrows_v.at[0], QNBATCH, gsem.at[0])
    lax.fori_loop(1, QN_CHUNKS, q_chunk_body, 0)
    qlast = QN_CHUNKS - 1
    drain_chunk(qrows_v.at[qlast % 2], QCHUNK, gsem.at[qlast % 2])
    pltpu.async_copy(qrows_v.at[qlast % 2],
                     q_out_hbm.at[pl.ds(base_q + qlast * QCHUNK, QCHUNK)], wsem)
    pltpu.make_async_copy(qrows_v.at[0], q_out_hbm.at[pl.ds(base_q, QCHUNK)],
                          wsem).wait()
    pltpu.make_async_copy(qrows_v.at[0], q_out_hbm.at[pl.ds(base_q, QCHUNK)],
                          wsem).wait()


def _sc_gather_cq(cand_idx, q_idx, table):
    mesh = plsc.VectorSubcoreMesh(core_axis_name="c", subcore_axis_name="s")
    kfn = functools.partial(
        pl.kernel,
        mesh=mesh,
        out_type=[
            jax.ShapeDtypeStruct((HB * LC, WE), jnp.float32),
            jax.ShapeDtypeStruct((B * LQ, WE), jnp.float32),
        ],
        scratch_types=[
            pltpu.VMEM((C_CHUNKS, SCHUNK), jnp.int32),
            pltpu.VMEM((QN_CHUNKS, QCHUNK), jnp.int32),
            pltpu.VMEM((2, SCHUNK, WE), jnp.float32),
            pltpu.VMEM((2, QCHUNK, WE), jnp.float32),
            pltpu.SemaphoreType.DMA((2,)),
            pltpu.SemaphoreType.DMA,
        ],
    )(_sc_gather_kernel_cq)
    return kfn(cand_idx, q_idx, table)


def _sc_gather_c(cand_idx, table):
    mesh = plsc.VectorSubcoreMesh(core_axis_name="c", subcore_axis_name="s")
    kfn = functools.partial(
        pl.kernel,
        mesh=mesh,
        out_type=jax.ShapeDtypeStruct((HB * LC, WE), jnp.float32),
        scratch_types=[
            pltpu.VMEM((C_CHUNKS, SCHUNK), jnp.int32),
            pltpu.VMEM((2, SCHUNK, WE), jnp.float32),
            pltpu.SemaphoreType.DMA((2,)),
            pltpu.SemaphoreType.DMA,
        ],
    )(_sc_gather_kernel_c)
    return kfn(cand_idx, table)


BB = 64


def _tc_body(qe_ref, cand_ref, ids_ref, wk_ref, w1q_ref, w1ct_ref,
             b1_ref, cb_ref, w2_ref, probs_out, idx_out):
    f32 = jnp.float32

    q2 = qe_ref[...].reshape(BB * LQ, WE)
    y0 = jnp.dot(q2, wk_ref[0], preferred_element_type=f32).reshape(BB, LQ, QE)
    y1 = jnp.dot(q2, wk_ref[1], preferred_element_type=f32).reshape(BB, LQ, QE)
    y2 = jnp.dot(q2, wk_ref[2], preferred_element_type=f32).reshape(BB, LQ, QE)

    acc = y0[:, 0, :] + y1[:, 1, :] + y2[:, 2, :]
    for t in range(1, LQ - KS + 1):
        acc = jnp.maximum(acc, y0[:, t, :] + y1[:, t + 1, :] + y2[:, t + 2, :])
    qemb = jnp.maximum(acc + cb_ref[...], 0.0)

    qw = jnp.dot(qemb, w1q_ref[...], preferred_element_type=f32) + b1_ref[...]
    qwt = jnp.transpose(qw)

    candt = jnp.transpose(cand_ref[...].reshape(BB * LC, WE))
    projt = jnp.dot(w1ct_ref[...], candt, preferred_element_type=f32)
    hh = jnp.tanh(projt.reshape(QE, BB, LC) + qwt[:, :, None])
    logits = jnp.sum(hh * w2_ref[...][:, :, None], axis=0)

    m0 = jnp.max(logits, axis=-1, keepdims=True)
    e = jnp.exp(logits - m0)
    probs = e / jnp.sum(e, axis=-1, keepdims=True)

    iota = lax.broadcasted_iota(jnp.int32, (BB, LC), 1)
    ids = ids_ref[...]
    cur = probs
    p_cols, i_cols = [], []
    for _ in range(TOPN):
        m = jnp.max(cur, axis=-1, keepdims=True)
        pos = jnp.min(jnp.where(cur == m, iota, LC), axis=-1, keepdims=True)
        onehot = iota == pos
        p_cols.append(m)
        i_cols.append(jnp.sum(jnp.where(onehot, ids, 0), axis=-1, keepdims=True))
        cur = jnp.where(onehot, -1.0, cur)
    probs_out[...] = jnp.concatenate(p_cols, axis=1)
    idx_out[...] = jnp.concatenate(i_cols, axis=1)


def _tc_call(qe3, cand3, ids, wk, w1q, w1ct, b1r, cbr, w2c, interpret=False):
    grid = (HB // BB,)
    full = lambda shape: pl.BlockSpec(shape, lambda i, s=len(shape): (0,) * s)
    return pl.pallas_call(
        _tc_body,
        grid=grid,
        in_specs=[
            pl.BlockSpec((BB, LQ, WE), lambda i: (i, 0, 0)),
            pl.BlockSpec((BB, LC, WE), lambda i: (i, 0, 0)),
            pl.BlockSpec((BB, LC), lambda i: (i, 0)),
            full((KS, WE, QE)),
            full((QE, QE)),
            full((QE, WE)),
            full((1, QE)),
            full((1, QE)),
            full((QE, 1)),
        ],
        out_specs=[
            pl.BlockSpec((BB, TOPN), lambda i: (i, 0)),
            pl.BlockSpec((BB, TOPN), lambda i: (i, 0)),
        ],
        out_shape=[
            jax.ShapeDtypeStruct((HB, TOPN), jnp.float32),
            jax.ShapeDtypeStruct((HB, TOPN), jnp.int32),
        ],
        interpret=interpret,
    )(qe3, cand3, ids, wk, w1q, w1ct, b1r, cbr, w2c)


def kernel(query_batch, cand_term_batch, table, conv_w, conv_b, W1, b1, W2):
    cb32 = cand_term_batch.astype(jnp.int32)
    cand_idx1 = cb32[:HB].reshape(NW, C_CHUNKS, SCHUNK)
    cand_idx2 = cb32[HB:].reshape(NW, C_CHUNKS, SCHUNK)
    q_idx = query_batch.reshape(NW, QN_CHUNKS, QCHUNK).astype(jnp.int32)
    cand_rows1, q_rows = _sc_gather_cq(cand_idx1, q_idx, table)
    cand_rows2 = _sc_gather_c(cand_idx2, table)

    wk = jnp.transpose(conv_w, (2, 1, 0))
    w1q = W1[:QE]
    w1ct = jnp.transpose(W1[QE:])
    b1r = b1.reshape(1, QE)
    cbr = conv_b.reshape(1, QE)
    qe3 = q_rows.reshape(B, LQ, WE)

    pA, iA = _tc_call(qe3[:HB], cand_rows1.reshape(HB, LC, WE),
                      cb32[:HB], wk, w1q, w1ct, b1r, cbr, W2)
    pB, iB = _tc_call(qe3[HB:], cand_rows2.reshape(HB, LC, WE),
                      cb32[HB:], wk, w1q, w1ct, b1r, cbr, W2)
    return (jnp.concatenate([pA, pB], axis=0), jnp.concatenate([iA, iB], axis=0))


# speedup vs baseline: 1.2743x; 1.2743x over previous
import functools

import jax
import jax.numpy as jnp
from jax import lax
from jax.experimental import pallas as pl
from jax.experimental.pallas import tpu as pltpu
from jax.experimental.pallas import tpu_sc as plsc

B = 1024
LQ = 20
LC = 200
WE = 64
QE = 64
KS = 3
TOPN = 10

NC = 2
NS = 16
NW = NC * NS

FIRE = 16
SCHUNK = 320
HB = B // 2
C_PER_W = (HB * LC) // NW
Q_PER_W = (B * LQ) // NW
C_CHUNKS = C_PER_W // SCHUNK
QCHUNK = 128
QN_CHUNKS = Q_PER_W // QCHUNK
NBATCH = SCHUNK // FIRE
QNBATCH = QCHUNK // FIRE
HALF_V = 500000


def _sc_gather_kernel_cq(cand_idx_hbm, q_idx_hbm, table_hbm,
                         cand_out_hbm, q_out_hbm,
                         idx_c_v, idx_q_v, rows_v, qrows_v, gsem, wsem):
    _sc_gather_body(cand_idx_hbm, q_idx_hbm, table_hbm, cand_out_hbm,
                    q_out_hbm, idx_c_v, idx_q_v, rows_v, qrows_v, gsem, wsem)


def _sc_gather_kernel_c(cand_idx_hbm, table_hbm, cand_out_hbm,
                        idx_c_v, rows_v, gsem, wsem):
    _sc_gather_body(cand_idx_hbm, None, table_hbm, cand_out_hbm,
                    None, idx_c_v, None, rows_v, None, gsem, wsem)


def _sc_gather_body(cand_idx_hbm, q_idx_hbm, table_hbm,
                    cand_out_hbm, q_out_hbm,
                    idx_c_v, idx_q_v, rows_v, qrows_v, gsem, wsem):
    wid = lax.axis_index("s") * NC + lax.axis_index("c")
    base_c = wid * C_PER_W
    base_q = wid * Q_PER_W

    pltpu.sync_copy(cand_idx_hbm.at[wid], idx_c_v)
    if q_idx_hbm is not None:
        pltpu.sync_copy(q_idx_hbm.at[wid], idx_q_v)

    def fire_chunk(idx_ref, buf, nbatch, sem):
        def body(g, c):
            vec = idx_ref[pl.ds(g * FIRE, FIRE)]
            for u in range(FIRE):
                r = vec[u]
                hi = (r >= HALF_V).astype(jnp.int32)
                rr = r - hi * HALF_V
                pltpu.async_copy(table_hbm.at[hi, pl.ds(rr, 1)],
                                 buf.at[pl.ds(g * FIRE + u, 1)], sem)
            return c
        lax.fori_loop(0, nbatch, body, 0)

    def drain_chunk(buf, nrows, sem):
        def body(g, c):
            for u in range(FIRE):
                pltpu.make_async_copy(table_hbm.at[0, pl.ds(0, 1)],
                                      buf.at[pl.ds(g * FIRE + u, 1)], sem).wait()
            return c
        lax.fori_loop(0, nrows // FIRE, body, 0)

    def chunk_body(ch, carry):
        @pl.when(ch >= 2)
        def _():
            pltpu.make_async_copy(
                rows_v.at[0],
                cand_out_hbm.at[pl.ds(base_c, SCHUNK)], wsem).wait()
        fire_chunk(idx_c_v.at[ch], rows_v.at[ch % 2], NBATCH, gsem.at[ch % 2])
        prev = ch - 1
        drain_chunk(rows_v.at[prev % 2], SCHUNK, gsem.at[prev % 2])
        pltpu.async_copy(rows_v.at[prev % 2],
                         cand_out_hbm.at[pl.ds(base_c + prev * SCHUNK, SCHUNK)],
                         wsem)
        return carry

    fire_chunk(idx_c_v.at[0], rows_v.at[0], NBATCH, gsem.at[0])
    lax.fori_loop(1, C_CHUNKS, chunk_body, 0)
    last = C_CHUNKS - 1
    drain_chunk(rows_v.at[last % 2], SCHUNK, gsem.at[last % 2])
    pltpu.async_copy(rows_v.at[last % 2],
                     cand_out_hbm.at[pl.ds(base_c + last * SCHUNK, SCHUNK)], wsem)
    pltpu.make_async_copy(rows_v.at[0], cand_out_hbm.at[pl.ds(base_c, SCHUNK)],
                          wsem).wait()
    pltpu.make_async_copy(rows_v.at[0], cand_out_hbm.at[pl.ds(base_c, SCHUNK)],
                          wsem).wait()

    if q_idx_hbm is None:
        return

    def q_chunk_body(ch, carry):
        @pl.when(ch >= 2)
        def _():
            pltpu.make_async_copy(
                qrows_v.at[0],
                q_out_hbm.at[pl.ds(base_q, QCHUNK)], wsem).wait()
        fire_chunk(idx_q_v.at[ch], qrows_v.at[ch % 2], QNBATCH, gsem.at[ch % 2])
        prev = ch - 1
        drain_chunk(qrows_v.at[prev % 2], QCHUNK, gsem.at[prev % 2])
        pltpu.async_copy(qrows_v.at[prev % 2],
                         q_out_hbm.at[pl.ds(base_q + prev * QCHUNK, QCHUNK)], wsem)
        return carry

    fire_chunk(idx_q_v.at[0], qrows_v.at[0], QNBATCH, gsem.at[0])
    lax.fori_loop(1, QN_CHUNKS, q_chunk_body, 0)
    qlast = QN_CHUNKS - 1
    drain_chunk(qrows_v.at[qlast % 2], QCHUNK, gsem.at[qlast % 2])
    pltpu.async_copy(qrows_v.at[qlast % 2],
                     q_out_hbm.at[pl.ds(base_q + qlast * QCHUNK, QCHUNK)], wsem)
    pltpu.make_async_copy(qrows_v.at[0], q_out_hbm.at[pl.ds(base_q, QCHUNK)],
                          wsem).wait()
    pltpu.make_async_copy(qrows_v.at[0], q_out_hbm.at[pl.ds(base_q, QCHUNK)],
                          wsem).wait()


def _sc_gather_cq(cand_idx, q_idx, table):
    mesh = plsc.VectorSubcoreMesh(core_axis_name="c", subcore_axis_name="s")
    kfn = functools.partial(
        pl.kernel,
        mesh=mesh,
        out_type=[
            jax.ShapeDtypeStruct((HB * LC, WE), jnp.float32),
            jax.ShapeDtypeStruct((B * LQ, WE), jnp.float32),
        ],
        scratch_types=[
            pltpu.VMEM((C_CHUNKS, SCHUNK), jnp.int32),
            pltpu.VMEM((QN_CHUNKS, QCHUNK), jnp.int32),
            pltpu.VMEM((2, SCHUNK, WE), jnp.float32),
            pltpu.VMEM((2, QCHUNK, WE), jnp.float32),
            pltpu.SemaphoreType.DMA((2,)),
            pltpu.SemaphoreType.DMA,
        ],
    )(_sc_gather_kernel_cq)
    return kfn(cand_idx, q_idx, table)


def _sc_gather_c(cand_idx, table):
    mesh = plsc.VectorSubcoreMesh(core_axis_name="c", subcore_axis_name="s")
    kfn = functools.partial(
        pl.kernel,
        mesh=mesh,
        out_type=jax.ShapeDtypeStruct((HB * LC, WE), jnp.float32),
        scratch_types=[
            pltpu.VMEM((C_CHUNKS, SCHUNK), jnp.int32),
            pltpu.VMEM((2, SCHUNK, WE), jnp.float32),
            pltpu.SemaphoreType.DMA((2,)),
            pltpu.SemaphoreType.DMA,
        ],
    )(_sc_gather_kernel_c)
    return kfn(cand_idx, table)


BB = 64


def _tc_body(qe_ref, cand_ref, ids_ref, wk_ref, w1q_ref, w1ct_ref,
             b1_ref, cb_ref, w2_ref, probs_out, idx_out):
    f32 = jnp.float32

    q2 = qe_ref[...].reshape(BB * LQ, WE)
    y0 = jnp.dot(q2, wk_ref[0], preferred_element_type=f32).reshape(BB, LQ, QE)
    y1 = jnp.dot(q2, wk_ref[1], preferred_element_type=f32).reshape(BB, LQ, QE)
    y2 = jnp.dot(q2, wk_ref[2], preferred_element_type=f32).reshape(BB, LQ, QE)

    acc = y0[:, 0, :] + y1[:, 1, :] + y2[:, 2, :]
    for t in range(1, LQ - KS + 1):
        acc = jnp.maximum(acc, y0[:, t, :] + y1[:, t + 1, :] + y2[:, t + 2, :])
    qemb = jnp.maximum(acc + cb_ref[...], 0.0)

    qw = jnp.dot(qemb, w1q_ref[...], preferred_element_type=f32) + b1_ref[...]
    qwt = jnp.transpose(qw)

    candt = jnp.transpose(cand_ref[...].reshape(BB * LC, WE))
    projt = jnp.dot(w1ct_ref[...], candt, preferred_element_type=f32)
    hh = jnp.tanh(projt.reshape(QE, BB, LC) + qwt[:, :, None])
    logits = jnp.sum(hh * w2_ref[...][:, :, None], axis=0)

    m0 = jnp.max(logits, axis=-1, keepdims=True)
    e = jnp.exp(logits - m0)
    probs = e / jnp.sum(e, axis=-1, keepdims=True)

    iota = lax.broadcasted_iota(jnp.int32, (BB, LC), 1)
    ids = ids_ref[...]
    cur = probs
    p_cols, i_cols = [], []
    for _ in range(TOPN):
        m = jnp.max(cur, axis=-1, keepdims=True)
        pos = jnp.min(jnp.where(cur == m, iota, LC), axis=-1, keepdims=True)
        onehot = iota == pos
        p_cols.append(m)
        i_cols.append(jnp.sum(jnp.where(onehot, ids, 0), axis=-1, keepdims=True))
        cur = jnp.where(onehot, -1.0, cur)
    probs_out[...] = jnp.concatenate(p_cols, axis=1)
    idx_out[...] = jnp.concatenate(i_cols, axis=1)


def _tc_call(qe3, cand3, ids, wk, w1q, w1ct, b1r, cbr, w2c, interpret=False):
    grid = (HB // BB,)
    full = lambda shape: pl.BlockSpec(shape, lambda i, s=len(shape): (0,) * s)
    return pl.pallas_call(
        _tc_body,
        grid=grid,
        in_specs=[
            pl.BlockSpec((BB, LQ, WE), lambda i: (i, 0, 0)),
            pl.BlockSpec((BB, LC, WE), lambda i: (i, 0, 0)),
            pl.BlockSpec((BB, LC), lambda i: (i, 0)),
            full((KS, WE, QE)),
            full((QE, QE)),
            full((QE, WE)),
            full((1, QE)),
            full((1, QE)),
            full((QE, 1)),
        ],
        out_specs=[
            pl.BlockSpec((BB, TOPN), lambda i: (i, 0)),
            pl.BlockSpec((BB, TOPN), lambda i: (i, 0)),
        ],
        out_shape=[
            jax.ShapeDtypeStruct((HB, TOPN), jnp.float32),
            jax.ShapeDtypeStruct((HB, TOPN), jnp.int32),
        ],
        interpret=interpret,
    )(qe3, cand3, ids, wk, w1q, w1ct, b1r, cbr, w2c)


def kernel(query_batch, cand_term_batch, table, conv_w, conv_b, W1, b1, W2):
    cb32 = cand_term_batch.astype(jnp.int32)
    cand_idx1 = cb32[:HB].reshape(NW, C_CHUNKS, SCHUNK)
    cand_idx2 = cb32[HB:].reshape(NW, C_CHUNKS, SCHUNK)
    q_idx = query_batch.reshape(NW, QN_CHUNKS, QCHUNK).astype(jnp.int32)
    tview = table.reshape(2, HALF_V, WE)
    cand_rows1, q_rows = _sc_gather_cq(cand_idx1, q_idx, tview)
    cand_rows2 = _sc_gather_c(cand_idx2, tview)

    wk = jnp.transpose(conv_w, (2, 1, 0))
    w1q = W1[:QE]
    w1ct = jnp.transpose(W1[QE:])
    b1r = b1.reshape(1, QE)
    cbr = conv_b.reshape(1, QE)
    qe3 = q_rows.reshape(B, LQ, WE)

    pA, iA = _tc_call(qe3[:HB], cand_rows1.reshape(HB, LC, WE),
                      cb32[:HB], wk, w1q, w1ct, b1r, cbr, W2)
    pB, iB = _tc_call(qe3[HB:], cand_rows2.reshape(HB, LC, WE),
                      cb32[HB:], wk, w1q, w1ct, b1r, cbr, W2)
    return (jnp.concatenate([pA, pB], axis=0), jnp.concatenate([iA, iB], axis=0))


# speedup vs baseline: 1.3610x; 1.0680x over previous
import functools

import jax
import jax.numpy as jnp
from jax import lax
from jax.experimental import pallas as pl
from jax.experimental.pallas import tpu as pltpu
from jax.experimental.pallas import tpu_sc as plsc

B = 1024
LQ = 20
LC = 200
WE = 64
QE = 64
KS = 3
TOPN = 10

NC = 2
NS = 16
NW = NC * NS

FIRE = 16
SCHUNK = 320
HB = B // 2
C_PER_W = (HB * LC) // NW
Q_PER_W = (B * LQ) // NW
C_CHUNKS = C_PER_W // SCHUNK
QCHUNK = 128
QN_CHUNKS = Q_PER_W // QCHUNK
NBATCH = SCHUNK // FIRE
QNBATCH = QCHUNK // FIRE
HALF_V = 500000


def _sc_gather_kernel_cq(cand_idx_hbm, q_idx_hbm, table_hbm,
                         cand_out_hbm, q_out_hbm,
                         idx_c_v, idx_q_v, rows_v, qrows_v, gsem, wsem):
    _sc_gather_body(cand_idx_hbm, q_idx_hbm, table_hbm, cand_out_hbm,
                    q_out_hbm, idx_c_v, idx_q_v, rows_v, qrows_v, gsem, wsem)


def _sc_gather_kernel_c(cand_idx_hbm, table_hbm, cand_out_hbm,
                        idx_c_v, rows_v, gsem, wsem):
    _sc_gather_body(cand_idx_hbm, None, table_hbm, cand_out_hbm,
                    None, idx_c_v, None, rows_v, None, gsem, wsem)


def _sc_gather_body(cand_idx_hbm, q_idx_hbm, table_hbm,
                    cand_out_hbm, q_out_hbm,
                    idx_c_v, idx_q_v, rows_v, qrows_v, gsem, wsem):
    wid = lax.axis_index("s") * NC + lax.axis_index("c")
    base_c = wid * C_PER_W
    base_q = wid * Q_PER_W

    pltpu.sync_copy(cand_idx_hbm.at[wid], idx_c_v)
    if q_idx_hbm is not None:
        pltpu.sync_copy(q_idx_hbm.at[wid], idx_q_v)

    def fire_chunk(idx_ref, buf, nbatch, sem):
        def body(g, c):
            vec = idx_ref[pl.ds(g * FIRE, FIRE)]
            for u in range(FIRE):
                r = vec[u]
                hi = (r >= HALF_V).astype(jnp.int32)
                rr = r - hi * HALF_V
                pltpu.async_copy(table_hbm.at[hi, pl.ds(rr, 1)],
                                 buf.at[pl.ds(g * FIRE + u, 1)], sem)
            return c
        lax.fori_loop(0, nbatch, body, 0)

    def drain_chunk(buf, nrows, sem):
        def body(g, c):
            for u in range(FIRE):
                pltpu.make_async_copy(table_hbm.at[0, pl.ds(0, 1)],
                                      buf.at[pl.ds(g * FIRE + u, 1)], sem).wait()
            return c
        lax.fori_loop(0, nrows // FIRE, body, 0)

    def chunk_body(ch, carry):
        @pl.when(ch >= 2)
        def _():
            pltpu.make_async_copy(
                rows_v.at[0],
                cand_out_hbm.at[pl.ds(base_c, SCHUNK)], wsem).wait()
        fire_chunk(idx_c_v.at[ch], rows_v.at[ch % 2], NBATCH, gsem.at[ch % 2])
        prev = ch - 1
        drain_chunk(rows_v.at[prev % 2], SCHUNK, gsem.at[prev % 2])
        pltpu.async_copy(rows_v.at[prev % 2],
                         cand_out_hbm.at[pl.ds(base_c + prev * SCHUNK, SCHUNK)],
                         wsem)
        return carry

    fire_chunk(idx_c_v.at[0], rows_v.at[0], NBATCH, gsem.at[0])
    lax.fori_loop(1, C_CHUNKS, chunk_body, 0)
    last = C_CHUNKS - 1
    drain_chunk(rows_v.at[last % 2], SCHUNK, gsem.at[last % 2])
    pltpu.async_copy(rows_v.at[last % 2],
                     cand_out_hbm.at[pl.ds(base_c + last * SCHUNK, SCHUNK)], wsem)
    pltpu.make_async_copy(rows_v.at[0], cand_out_hbm.at[pl.ds(base_c, SCHUNK)],
                          wsem).wait()
    pltpu.make_async_copy(rows_v.at[0], cand_out_hbm.at[pl.ds(base_c, SCHUNK)],
                          wsem).wait()

    if q_idx_hbm is None:
        return

    def q_chunk_body(ch, carry):
        @pl.when(ch >= 2)
        def _():
            pltpu.make_async_copy(
                qrows_v.at[0],
                q_out_hbm.at[pl.ds(base_q, QCHUNK)], wsem).wait()
        fire_chunk(idx_q_v.at[ch], qrows_v.at[ch % 2], QNBATCH, gsem.at[ch % 2])
        prev = ch - 1
        drain_chunk(qrows_v.at[prev % 2], QCHUNK, gsem.at[prev % 2])
        pltpu.async_copy(qrows_v.at[prev % 2],
                         q_out_hbm.at[pl.ds(base_q + prev * QCHUNK, QCHUNK)], wsem)
        return carry

    fire_chunk(idx_q_v.at[0], qrows_v.at[0], QNBATCH, gsem.at[0])
    lax.fori_loop(1, QN_CHUNKS, q_chunk_body, 0)
    qlast = QN_CHUNKS - 1
    drain_chunk(qrows_v.at[qlast % 2], QCHUNK, gsem.at[qlast % 2])
    pltpu.async_copy(qrows_v.at[qlast % 2],
                     q_out_hbm.at[pl.ds(base_q + qlast * QCHUNK, QCHUNK)], wsem)
    pltpu.make_async_copy(qrows_v.at[0], q_out_hbm.at[pl.ds(base_q, QCHUNK)],
                          wsem).wait()
    pltpu.make_async_copy(qrows_v.at[0], q_out_hbm.at[pl.ds(base_q, QCHUNK)],
                          wsem).wait()


def _sc_gather_cq(cand_idx, q_idx, table):
    mesh = plsc.VectorSubcoreMesh(core_axis_name="c", subcore_axis_name="s")
    kfn = functools.partial(
        pl.kernel,
        mesh=mesh,
        out_type=[
            jax.ShapeDtypeStruct((HB * LC, WE), jnp.float32),
            jax.ShapeDtypeStruct((B * LQ, WE), jnp.float32),
        ],
        scratch_types=[
            pltpu.VMEM((C_CHUNKS, SCHUNK), jnp.int32),
            pltpu.VMEM((QN_CHUNKS, QCHUNK), jnp.int32),
            pltpu.VMEM((2, SCHUNK, WE), jnp.float32),
            pltpu.VMEM((2, QCHUNK, WE), jnp.float32),
            pltpu.SemaphoreType.DMA((2,)),
            pltpu.SemaphoreType.DMA,
        ],
    )(_sc_gather_kernel_cq)
    return kfn(cand_idx, q_idx, table)


def _sc_gather_c(cand_idx, table):
    mesh = plsc.VectorSubcoreMesh(core_axis_name="c", subcore_axis_name="s")
    kfn = functools.partial(
        pl.kernel,
        mesh=mesh,
        out_type=jax.ShapeDtypeStruct((HB * LC, WE), jnp.float32),
        scratch_types=[
            pltpu.VMEM((C_CHUNKS, SCHUNK), jnp.int32),
            pltpu.VMEM((2, SCHUNK, WE), jnp.float32),
            pltpu.SemaphoreType.DMA((2,)),
            pltpu.SemaphoreType.DMA,
        ],
    )(_sc_gather_kernel_c)
    return kfn(cand_idx, table)


BB = 64


def _tc_body(qe_ref, cand_ref, ids_ref, wk_ref, w1q_ref, w1ct_ref,
             b1_ref, cb_ref, w2_ref, probs_out, idx_out):
    f32 = jnp.float32

    q2 = qe_ref[...]
    y0 = jnp.dot(q2, wk_ref[0], preferred_element_type=f32).reshape(BB, LQ, QE)
    y1 = jnp.dot(q2, wk_ref[1], preferred_element_type=f32).reshape(BB, LQ, QE)
    y2 = jnp.dot(q2, wk_ref[2], preferred_element_type=f32).reshape(BB, LQ, QE)

    acc = y0[:, 0, :] + y1[:, 1, :] + y2[:, 2, :]
    for t in range(1, LQ - KS + 1):
        acc = jnp.maximum(acc, y0[:, t, :] + y1[:, t + 1, :] + y2[:, t + 2, :])
    qemb = jnp.maximum(acc + cb_ref[...], 0.0)

    qw = jnp.dot(qemb, w1q_ref[...], preferred_element_type=f32) + b1_ref[...]
    qwt = jnp.transpose(qw)

    candt = jnp.transpose(cand_ref[...].reshape(BB * LC, WE))
    projt = jnp.dot(w1ct_ref[...], candt, preferred_element_type=f32)
    hh = jnp.tanh(projt.reshape(QE, BB, LC) + qwt[:, :, None])
    logits = jnp.sum(hh * w2_ref[...][:, :, None], axis=0)

    m0 = jnp.max(logits, axis=-1, keepdims=True)
    e = jnp.exp(logits - m0)
    probs = e / jnp.sum(e, axis=-1, keepdims=True)

    iota = lax.broadcasted_iota(jnp.int32, (BB, LC), 1)
    ids = ids_ref[...]
    cur = probs
    p_cols, i_cols = [], []
    for _ in range(TOPN):
        m = jnp.max(cur, axis=-1, keepdims=True)
        pos = jnp.min(jnp.where(cur == m, iota, LC), axis=-1, keepdims=True)
        onehot = iota == pos
        p_cols.append(m)
        i_cols.append(jnp.sum(jnp.where(onehot, ids, 0), axis=-1, keepdims=True))
        cur = jnp.where(onehot, -1.0, cur)
    probs_out[...] = jnp.concatenate(p_cols, axis=1)
    idx_out[...] = jnp.concatenate(i_cols, axis=1)


def _tc_call(qe3, cand3, ids, wk, w1q, w1ct, b1r, cbr, w2c, interpret=False):
    grid = (HB // BB,)
    full = lambda shape: pl.BlockSpec(shape, lambda i, s=len(shape): (0,) * s)
    return pl.pallas_call(
        _tc_body,
        grid=grid,
        in_specs=[
            pl.BlockSpec((BB * LQ, WE), lambda i: (i, 0)),
            pl.BlockSpec((BB, LC, WE), lambda i: (i, 0, 0)),
            pl.BlockSpec((BB, LC), lambda i: (i, 0)),
            full((KS, WE, QE)),
            full((QE, QE)),
            full((QE, WE)),
            full((1, QE)),
            full((1, QE)),
            full((QE, 1)),
        ],
        out_specs=[
            pl.BlockSpec((BB, TOPN), lambda i: (i, 0)),
            pl.BlockSpec((BB, TOPN), lambda i: (i, 0)),
        ],
        out_shape=[
            jax.ShapeDtypeStruct((HB, TOPN), jnp.float32),
            jax.ShapeDtypeStruct((HB, TOPN), jnp.int32),
        ],
        interpret=interpret,
    )(qe3, cand3, ids, wk, w1q, w1ct, b1r, cbr, w2c)


def kernel(query_batch, cand_term_batch, table, conv_w, conv_b, W1, b1, W2):
    cb32 = cand_term_batch.astype(jnp.int32)
    cand_idx1 = cb32[:HB].reshape(NW, C_CHUNKS, SCHUNK)
    cand_idx2 = cb32[HB:].reshape(NW, C_CHUNKS, SCHUNK)
    q_idx = query_batch.reshape(NW, QN_CHUNKS, QCHUNK).astype(jnp.int32)
    tview = table.reshape(2, HALF_V, WE)
    cand_rows1, q_rows = _sc_gather_cq(cand_idx1, q_idx, tview)
    cand_rows2 = _sc_gather_c(cand_idx2, tview)

    wk = jnp.transpose(conv_w, (2, 1, 0))
    w1q = W1[:QE]
    w1ct = jnp.transpose(W1[QE:])
    b1r = b1.reshape(1, QE)
    cbr = conv_b.reshape(1, QE)
    pA, iA = _tc_call(q_rows[:HB * LQ], cand_rows1.reshape(HB, LC, WE),
                      cb32[:HB], wk, w1q, w1ct, b1r, cbr, W2)
    pB, iB = _tc_call(q_rows[HB * LQ:], cand_rows2.reshape(HB, LC, WE),
                      cb32[HB:], wk, w1q, w1ct, b1r, cbr, W2)
    return (jnp.concatenate([pA, pB], axis=0), jnp.concatenate([iA, iB], axis=0))
